# baseline (speedup 1.0000x reference)
import jax
import jax.numpy as jnp
from jax import lax
from jax.experimental import pallas as pl
from jax.experimental.pallas import tpu as pltpu
from jax.experimental.pallas import tpu_sc as plsc

N = 10000
D = 128
E = 320000
NC, NS, L = 2, 16, 16
NW = NC * NS
CHUNK = 128
CPW = -(-E // (NW * CHUNK))
EPW = CPW * CHUNK
E_PAD = NW * EPW
NPAD = 10240
ZR = NPAD // NS
NG = 4
NK = 2
NBUF = NG + NK
NG2 = 3
NK2 = 3
NBUF2 = NG2 + NK2

_MESH = plsc.VectorSubcoreMesh(core_axis_name="c", subcore_axis_name="s")


def _sc_agg_l1_body(x_hbm, src_hbm, dst_hbm, z128_hbm, z16_hbm, ones_hbm,
                    agg_hbm, cnt_hbm,
                    src_v, dst_v, rows_v, ones_v, acc, acc_cnt, gsem, ssem):
    c = lax.axis_index("c")
    s = lax.axis_index("s")
    wid = s * NC + c
    pltpu.sync_copy(z128_hbm, acc.at[pl.ds(s * ZR, ZR)])
    pltpu.sync_copy(z16_hbm, acc_cnt.at[pl.ds(s * ZR, ZR)])
    pltpu.sync_copy(src_hbm.at[wid], src_v)
    pltpu.sync_copy(dst_hbm.at[wid], dst_v)
    pltpu.sync_copy(ones_hbm, ones_v)
    plsc.subcore_barrier()

    for b in range(NG):
        pltpu.async_copy(x_hbm.at[src_v.at[b]], rows_v.at[b], gsem)

    def step(j, carry):
        pltpu.make_async_copy(x_hbm.at[src_v.at[j]],
                              rows_v.at[j % NBUF], gsem).wait()

        @pl.when(j >= NK)
        def _():
            pltpu.make_async_copy(rows_v.at[(j - NK) % NBUF],
                                  acc.at[dst_v.at[j - NK]], ssem).wait()

        @pl.when(j + NG < CPW)
        def _():
            pltpu.async_copy(x_hbm.at[src_v.at[j + NG]],
                             rows_v.at[(j + NG) % NBUF], gsem)

        pltpu.async_copy(rows_v.at[j % NBUF], acc.at[dst_v.at[j]], ssem,
                         add=True)
        pltpu.sync_copy(ones_v, acc_cnt.at[dst_v.at[j]], add=True)
        return carry

    lax.fori_loop(0, CPW, step, 0)
    for t in range(CPW - NK, CPW):
        pltpu.make_async_copy(rows_v.at[t % NBUF],
                              acc.at[dst_v.at[t]], ssem).wait()

    plsc.subcore_barrier()
    base = s * ZR
    pltpu.sync_copy(acc.at[pl.ds(base, ZR)], agg_hbm.at[c, pl.ds(base, ZR)])
    pltpu.sync_copy(acc_cnt.at[pl.ds(base, ZR)], cnt_hbm.at[c, pl.ds(base, ZR)])


_sc_agg_l1 = pl.kernel(
    _sc_agg_l1_body,
    out_type=(
        jax.ShapeDtypeStruct((NC, NPAD, D), jnp.bfloat16),
        jax.ShapeDtypeStruct((NC, NPAD, L), jnp.float32),
    ),
    mesh=_MESH,
    scratch_types=[
        pltpu.VMEM((CPW, CHUNK), jnp.int32),
        pltpu.VMEM((CPW, CHUNK), jnp.int32),
        pltpu.VMEM((NBUF, CHUNK, D), jnp.bfloat16),
        pltpu.VMEM((CHUNK, L), jnp.float32),
        pltpu.VMEM_SHARED((NPAD, D), jnp.bfloat16),
        pltpu.VMEM_SHARED((NPAD, L), jnp.float32),
        pltpu.SemaphoreType.DMA,
        pltpu.SemaphoreType.DMA,
    ],
    compiler_params=pltpu.CompilerParams(use_tc_tiling_on_sc=False),
)


def _sc_agg_l2_body(t_hbm, src_hbm, dst_hbm, z16_hbm,
                    agg_hbm,
                    src_v, dst_v, rows_v, acc, gsem, ssem):
    c = lax.axis_index("c")
    s = lax.axis_index("s")
    wid = s * NC + c
    pltpu.sync_copy(z16_hbm, acc.at[pl.ds(s * ZR, ZR)])
    pltpu.sync_copy(src_hbm.at[wid], src_v)
    pltpu.sync_copy(dst_hbm.at[wid], dst_v)
    plsc.subcore_barrier()

    for b in range(NG2):
        pltpu.async_copy(t_hbm.at[src_v.at[b]], rows_v.at[b], gsem)

    def step(j, carry):
        pltpu.make_async_copy(t_hbm.at[src_v.at[j]],
                              rows_v.at[j % NBUF2], gsem).wait()

        @pl.when(j >= NK2)
        def _():
            pltpu.make_async_copy(rows_v.at[(j - NK2) % NBUF2],
                                  acc.at[dst_v.at[j - NK2]], ssem).wait()

        @pl.when(j + NG2 < CPW)
        def _():
            pltpu.async_copy(t_hbm.at[src_v.at[j + NG2]],
                             rows_v.at[(j + NG2) % NBUF2], gsem)

        pltpu.async_copy(rows_v.at[j % NBUF2], acc.at[dst_v.at[j]], ssem,
                         add=True)
        return carry

    lax.fori_loop(0, CPW, step, 0)
    for t in range(CPW - NK2, CPW):
        pltpu.make_async_copy(rows_v.at[t % NBUF2],
                              acc.at[dst_v.at[t]], ssem).wait()
    plsc.subcore_barrier()
    base = s * ZR
    pltpu.sync_copy(acc.at[pl.ds(base, ZR)], agg_hbm.at[c, pl.ds(base, ZR)])


_sc_agg_l2 = pl.kernel(
    _sc_agg_l2_body,
    out_type=jax.ShapeDtypeStruct((NC, NPAD, L), jnp.float32),
    mesh=_MESH,
    scratch_types=[
        pltpu.VMEM((CPW, CHUNK), jnp.int32),
        pltpu.VMEM((CPW, CHUNK), jnp.int32),
        pltpu.VMEM((NBUF2, CHUNK, L), jnp.float32),
        pltpu.VMEM_SHARED((NPAD, L), jnp.float32),
        pltpu.SemaphoreType.DMA,
        pltpu.SemaphoreType.DMA,
    ],
    compiler_params=pltpu.CompilerParams(use_tc_tiling_on_sc=False),
)


def _matmul_t(a, w):
    return lax.dot_general(a, w, (((1,), (1,)), ((), ())),
                           preferred_element_type=jnp.float32)


BR = 1000


def _tc_layer1_body(aggp, cntp, x, w1l, w1r, b1, w2lp, w2rp, b2p,
                    h2_out, hr_out):
    agg = (aggp[0].astype(jnp.float32) + aggp[1].astype(jnp.float32))
    cnt = cntp[0, :, 0:1] + cntp[1, :, 0:1]
    mean = agg / jnp.maximum(cnt, 1.0)
    h = jnp.maximum(_matmul_t(mean, w1l[...]) + b1[...] +
                    _matmul_t(x[...], w1r[...]), 0.0)
    h2_out[...] = _matmul_t(h, w2lp[...])
    hr_out[...] = _matmul_t(h, w2rp[...]) + b2p[...]


def _tc_layer1(agg, cnt, x, w1l, w1r, b1, w2lp, w2rp, b2p):
    grid = N // BR
    full = lambda shape: pl.BlockSpec(shape, lambda i: (0,) * len(shape))
    return pl.pallas_call(
        _tc_layer1_body,
        grid=(grid,),
        in_specs=[
            pl.BlockSpec((NC, BR, D), lambda i: (0, i, 0)),
            pl.BlockSpec((NC, BR, L), lambda i: (0, i, 0)),
            pl.BlockSpec((BR, D), lambda i: (i, 0)),
            full((D, D)),
            full((D, D)),
            full((1, D)),
            full((L, D)),
            full((L, D)),
            full((1, L)),
        ],
        out_specs=[
            pl.BlockSpec((BR, L), lambda i: (i, 0)),
            pl.BlockSpec((BR, L), lambda i: (i, 0)),
        ],
        out_shape=[
            jax.ShapeDtypeStruct((N, L), jnp.float32),
            jax.ShapeDtypeStruct((N, L), jnp.float32),
        ],
    )(agg, cnt, x, w1l, w1r, b1, w2lp, w2rp, b2p)


def _tc_layer2_body(a2p, cntp, hr, out):
    a = a2p[0] + a2p[1]
    cnt = cntp[0, :, 0:1] + cntp[1, :, 0:1]
    mean2 = a / jnp.maximum(cnt, 1.0)
    out[...] = mean2[:, 0:2] + hr[:, 0:2]


def _tc_layer2(agg2, cnt, hr):
    grid = N // BR
    return pl.pallas_call(
        _tc_layer2_body,
        grid=(grid,),
        in_specs=[
            pl.BlockSpec((NC, BR, L), lambda i: (0, i, 0)),
            pl.BlockSpec((NC, BR, L), lambda i: (0, i, 0)),
            pl.BlockSpec((BR, L), lambda i: (i, 0)),
        ],
        out_specs=pl.BlockSpec((BR, 2), lambda i: (i, 0)),
        out_shape=jax.ShapeDtypeStruct((N, 2), jnp.float32),
    )(agg2, cnt, hr)


def kernel(x, edge_index, W1_l, W1_r, b1, W2_l, W2_r, b2):
    src = edge_index[0].astype(jnp.int32)
    dst = edge_index[1].astype(jnp.int32)
    pad = E_PAD - E
    junk = N + (jnp.arange(pad, dtype=jnp.int32) % (NPAD - N))
    src3 = jnp.concatenate([src, jnp.zeros((pad,), jnp.int32)]
                           ).reshape(NW, CPW, CHUNK)
    dst3 = jnp.concatenate([dst, junk]).reshape(NW, CPW, CHUNK)
    z128 = jnp.zeros((ZR, D), jnp.bfloat16)
    z16 = jnp.zeros((ZR, L), jnp.float32)
    ones16 = jnp.ones((CHUNK, L), jnp.float32)
    xbf = x.astype(jnp.bfloat16)

    agg, cnt = _sc_agg_l1(xbf, src3, dst3, z128, z16, ones16)

    w2lp = jnp.zeros((L, D), jnp.float32).at[0:2].set(W2_l)
    w2rp = jnp.zeros((L, D), jnp.float32).at[0:2].set(W2_r)
    b2p = jnp.zeros((1, L), jnp.float32).at[0, 0:2].set(b2)
    h2, hr = _tc_layer1(agg, cnt, x, W1_l, W1_r, b1.reshape(1, D),
                        w2lp, w2rp, b2p)

    agg2 = _sc_agg_l2(h2, src3, dst3, z16)
    return _tc_layer2(agg2, cnt, hr)


# speedup vs baseline: 1.0013x; 1.0013x over previous
import jax
import jax.numpy as jnp
from jax import lax
from jax.experimental import pallas as pl
from jax.experimental.pallas import tpu as pltpu
from jax.experimental.pallas import tpu_sc as plsc

N = 10000
D = 128
E = 320000
NC, NS, L = 2, 16, 16
NW = NC * NS
CHUNK = 128
CPW = -(-E // (NW * CHUNK))
EPW = CPW * CHUNK
E_PAD = NW * EPW
NPAD = 10240
ZR = NPAD // NS
NG = 5
NK = 1
NBUF = NG + NK
NG2 = 3
NK2 = 3
NBUF2 = NG2 + NK2

_MESH = plsc.VectorSubcoreMesh(core_axis_name="c", subcore_axis_name="s")


def _sc_agg_l1_body(x_hbm, src_hbm, dst_hbm, z128_hbm, z16_hbm, ones_hbm,
                    agg_hbm, cnt_hbm,
                    src_v, dst_v, rows_v, ones_v, acc, acc_cnt, gsem, ssem):
    c = lax.axis_index("c")
    s = lax.axis_index("s")
    wid = s * NC + c
    pltpu.sync_copy(z128_hbm, acc.at[pl.ds(s * ZR, ZR)])
    pltpu.sync_copy(z16_hbm, acc_cnt.at[pl.ds(s * ZR, ZR)])
    pltpu.sync_copy(src_hbm.at[wid], src_v)
    pltpu.sync_copy(dst_hbm.at[wid], dst_v)
    pltpu.sync_copy(ones_hbm, ones_v)
    plsc.subcore_barrier()

    for b in range(NG):
        pltpu.async_copy(x_hbm.at[src_v.at[b]], rows_v.at[b], gsem)

    def step(j, carry):
        pltpu.make_async_copy(x_hbm.at[src_v.at[j]],
                              rows_v.at[j % NBUF], gsem).wait()

        @pl.when(j >= NK)
        def _():
            pltpu.make_async_copy(rows_v.at[(j - NK) % NBUF],
                                  acc.at[dst_v.at[j - NK]], ssem).wait()

        @pl.when(j + NG < CPW)
        def _():
            pltpu.async_copy(x_hbm.at[src_v.at[j + NG]],
                             rows_v.at[(j + NG) % NBUF], gsem)

        pltpu.async_copy(rows_v.at[j % NBUF], acc.at[dst_v.at[j]], ssem,
                         add=True)
        pltpu.sync_copy(ones_v, acc_cnt.at[dst_v.at[j]], add=True)
        return carry

    lax.fori_loop(0, CPW, step, 0)
    for t in range(CPW - NK, CPW):
        pltpu.make_async_copy(rows_v.at[t % NBUF],
                              acc.at[dst_v.at[t]], ssem).wait()

    plsc.subcore_barrier()
    base = s * ZR
    pltpu.sync_copy(acc.at[pl.ds(base, ZR)], agg_hbm.at[c, pl.ds(base, ZR)])
    pltpu.sync_copy(acc_cnt.at[pl.ds(base, ZR)], cnt_hbm.at[c, pl.ds(base, ZR)])


_sc_agg_l1 = pl.kernel(
    _sc_agg_l1_body,
    out_type=(
        jax.ShapeDtypeStruct((NC, NPAD, D), jnp.bfloat16),
        jax.ShapeDtypeStruct((NC, NPAD, L), jnp.float32),
    ),
    mesh=_MESH,
    scratch_types=[
        pltpu.VMEM((CPW, CHUNK), jnp.int32),
        pltpu.VMEM((CPW, CHUNK), jnp.int32),
        pltpu.VMEM((NBUF, CHUNK, D), jnp.bfloat16),
        pltpu.VMEM((CHUNK, L), jnp.float32),
        pltpu.VMEM_SHARED((NPAD, D), jnp.bfloat16),
        pltpu.VMEM_SHARED((NPAD, L), jnp.float32),
        pltpu.SemaphoreType.DMA,
        pltpu.SemaphoreType.DMA,
    ],
    compiler_params=pltpu.CompilerParams(use_tc_tiling_on_sc=False),
)


def _sc_agg_l2_body(t_hbm, src_hbm, dst_hbm, z16_hbm,
                    agg_hbm,
                    src_v, dst_v, rows_v, acc, gsem, ssem):
    c = lax.axis_index("c")
    s = lax.axis_index("s")
    wid = s * NC + c
    pltpu.sync_copy(z16_hbm, acc.at[pl.ds(s * ZR, ZR)])
    pltpu.sync_copy(src_hbm.at[wid], src_v)
    pltpu.sync_copy(dst_hbm.at[wid], dst_v)
    plsc.subcore_barrier()

    for b in range(NG2):
        pltpu.async_copy(t_hbm.at[src_v.at[b]], rows_v.at[b], gsem)

    def step(j, carry):
        pltpu.make_async_copy(t_hbm.at[src_v.at[j]],
                              rows_v.at[j % NBUF2], gsem).wait()

        @pl.when(j >= NK2)
        def _():
            pltpu.make_async_copy(rows_v.at[(j - NK2) % NBUF2],
                                  acc.at[dst_v.at[j - NK2]], ssem).wait()

        @pl.when(j + NG2 < CPW)
        def _():
            pltpu.async_copy(t_hbm.at[src_v.at[j + NG2]],
                             rows_v.at[(j + NG2) % NBUF2], gsem)

        pltpu.async_copy(rows_v.at[j % NBUF2], acc.at[dst_v.at[j]], ssem,
                         add=True)
        return carry

    lax.fori_loop(0, CPW, step, 0)
    for t in range(CPW - NK2, CPW):
        pltpu.make_async_copy(rows_v.at[t % NBUF2],
                              acc.at[dst_v.at[t]], ssem).wait()
    plsc.subcore_barrier()
    base = s * ZR
    pltpu.sync_copy(acc.at[pl.ds(base, ZR)], agg_hbm.at[c, pl.ds(base, ZR)])


_sc_agg_l2 = pl.kernel(
    _sc_agg_l2_body,
    out_type=jax.ShapeDtypeStruct((NC, NPAD, L), jnp.float32),
    mesh=_MESH,
    scratch_types=[
        pltpu.VMEM((CPW, CHUNK), jnp.int32),
        pltpu.VMEM((CPW, CHUNK), jnp.int32),
        pltpu.VMEM((NBUF2, CHUNK, L), jnp.float32),
        pltpu.VMEM_SHARED((NPAD, L), jnp.float32),
        pltpu.SemaphoreType.DMA,
        pltpu.SemaphoreType.DMA,
    ],
    compiler_params=pltpu.CompilerParams(use_tc_tiling_on_sc=False),
)


def _matmul_t(a, w):
    return lax.dot_general(a, w, (((1,), (1,)), ((), ())),
                           preferred_element_type=jnp.float32)


BR = 1000


def _tc_layer1_body(aggp, cntp, x, w1l, w1r, b1, w2lp, w2rp, b2p,
                    h2_out, hr_out):
    agg = (aggp[0].astype(jnp.float32) + aggp[1].astype(jnp.float32))
    cnt = cntp[0, :, 0:1] + cntp[1, :, 0:1]
    mean = agg / jnp.maximum(cnt, 1.0)
    h = jnp.maximum(_matmul_t(mean, w1l[...]) + b1[...] +
                    _matmul_t(x[...], w1r[...]), 0.0)
    h2_out[...] = _matmul_t(h, w2lp[...])
    hr_out[...] = _matmul_t(h, w2rp[...]) + b2p[...]


def _tc_layer1(agg, cnt, x, w1l, w1r, b1, w2lp, w2rp, b2p):
    grid = N // BR
    full = lambda shape: pl.BlockSpec(shape, lambda i: (0,) * len(shape))
    return pl.pallas_call(
        _tc_layer1_body,
        grid=(grid,),
        in_specs=[
            pl.BlockSpec((NC, BR, D), lambda i: (0, i, 0)),
            pl.BlockSpec((NC, BR, L), lambda i: (0, i, 0)),
            pl.BlockSpec((BR, D), lambda i: (i, 0)),
            full((D, D)),
            full((D, D)),
            full((1, D)),
            full((L, D)),
            full((L, D)),
            full((1, L)),
        ],
        out_specs=[
            pl.BlockSpec((BR, L), lambda i: (i, 0)),
            pl.BlockSpec((BR, L), lambda i: (i, 0)),
        ],
        out_shape=[
            jax.ShapeDtypeStruct((N, L), jnp.float32),
            jax.ShapeDtypeStruct((N, L), jnp.float32),
        ],
    )(agg, cnt, x, w1l, w1r, b1, w2lp, w2rp, b2p)


def _tc_layer2_body(a2p, cntp, hr, out):
    a = a2p[0] + a2p[1]
    cnt = cntp[0, :, 0:1] + cntp[1, :, 0:1]
    mean2 = a / jnp.maximum(cnt, 1.0)
    out[...] = mean2[:, 0:2] + hr[:, 0:2]


def _tc_layer2(agg2, cnt, hr):
    grid = N // BR
    return pl.pallas_call(
        _tc_layer2_body,
        grid=(grid,),
        in_specs=[
            pl.BlockSpec((NC, BR, L), lambda i: (0, i, 0)),
            pl.BlockSpec((NC, BR, L), lambda i: (0, i, 0)),
            pl.BlockSpec((BR, L), lambda i: (i, 0)),
        ],
        out_specs=pl.BlockSpec((BR, 2), lambda i: (i, 0)),
        out_shape=jax.ShapeDtypeStruct((N, 2), jnp.float32),
    )(agg2, cnt, hr)


def kernel(x, edge_index, W1_l, W1_r, b1, W2_l, W2_r, b2):
    src = edge_index[0].astype(jnp.int32)
    dst = edge_index[1].astype(jnp.int32)
    pad = E_PAD - E
    junk = N + (jnp.arange(pad, dtype=jnp.int32) % (NPAD - N))
    src3 = jnp.concatenate([src, jnp.zeros((pad,), jnp.int32)]
                           ).reshape(NW, CPW, CHUNK)
    dst3 = jnp.concatenate([dst, junk]).reshape(NW, CPW, CHUNK)
    z128 = jnp.zeros((ZR, D), jnp.bfloat16)
    z16 = jnp.zeros((ZR, L), jnp.float32)
    ones16 = jnp.ones((CHUNK, L), jnp.float32)
    xbf = x.astype(jnp.bfloat16)

    agg, cnt = _sc_agg_l1(xbf, src3, dst3, z128, z16, ones16)

    w2lp = jnp.zeros((L, D), jnp.float32).at[0:2].set(W2_l)
    w2rp = jnp.zeros((L, D), jnp.float32).at[0:2].set(W2_r)
    b2p = jnp.zeros((1, L), jnp.float32).at[0, 0:2].set(b2)
    h2, hr = _tc_layer1(agg, cnt, x, W1_l, W1_r, b1.reshape(1, D),
                        w2lp, w2rp, b2p)

    agg2 = _sc_agg_l2(h2, src3, dst3, z16)
    return _tc_layer2(agg2, cnt, hr)


# speedup vs baseline: 1.0486x; 1.0472x over previous
import jax
import jax.numpy as jnp
from jax import lax
from jax.experimental import pallas as pl
from jax.experimental.pallas import tpu as pltpu
from jax.experimental.pallas import tpu_sc as plsc

N = 10000
D = 128
E = 320000
NC, NS, L = 2, 16, 16
NW = NC * NS
CHUNK = 128
CPT = -(-E // (NS * CHUNK))
EPT = CPT * CHUNK
E_PAD_T = NS * EPT
NPAD = 10240
ZR = NPAD // NS
NG = 3
NK = 1
NBUF = NG + NK
NG2 = 3
NK2 = 3
NBUF2 = NG2 + NK2

_MESH = plsc.VectorSubcoreMesh(core_axis_name="c", subcore_axis_name="s")


def _sc_agg_l1_body(xlo_hbm, xhi_hbm, src_hbm, dst_hbm, z64_hbm, z16_hbm,
                    ones_hbm,
                    agg_hbm, cnt_hbm,
                    src_v, dst_v, rows_v, ones_v, acc, acc_cnt, gsem, ssem):
    c = lax.axis_index("c")
    s = lax.axis_index("s")
    pltpu.sync_copy(z64_hbm, acc.at[pl.ds(s * ZR, ZR)])
    pltpu.sync_copy(z16_hbm, acc_cnt.at[pl.ds(s * ZR, ZR)])
    pltpu.sync_copy(src_hbm.at[s], src_v)
    pltpu.sync_copy(dst_hbm.at[s], dst_v)
    pltpu.sync_copy(ones_hbm, ones_v)
    plsc.subcore_barrier()

    def run(x_hbm, parity):
        for b in range(NG):
            pltpu.async_copy(x_hbm.at[src_v.at[b]], rows_v.at[b], gsem)

        def step(j, carry):
            pltpu.make_async_copy(x_hbm.at[src_v.at[j]],
                                  rows_v.at[j % NBUF], gsem).wait()

            @pl.when(j >= NK)
            def _():
                pltpu.make_async_copy(rows_v.at[(j - NK) % NBUF],
                                      acc.at[dst_v.at[j - NK]], ssem).wait()

            @pl.when(j + NG < CPT)
            def _():
                pltpu.async_copy(x_hbm.at[src_v.at[j + NG]],
                                 rows_v.at[(j + NG) % NBUF], gsem)

            pltpu.async_copy(rows_v.at[j % NBUF], acc.at[dst_v.at[j]], ssem,
                             add=True)

            @pl.when(lax.rem(j, 2) == parity)
            def _():
                pltpu.sync_copy(ones_v, acc_cnt.at[dst_v.at[j]], add=True)
            return carry

        lax.fori_loop(0, CPT, step, 0)
        for t in range(CPT - NK, CPT):
            pltpu.make_async_copy(rows_v.at[t % NBUF],
                                  acc.at[dst_v.at[t]], ssem).wait()

    @pl.when(c == 0)
    def _():
        run(xlo_hbm, 0)

    @pl.when(c == 1)
    def _():
        run(xhi_hbm, 1)

    plsc.subcore_barrier()
    base = s * ZR
    pltpu.sync_copy(acc.at[pl.ds(base, ZR)], agg_hbm.at[c, pl.ds(base, ZR)])
    pltpu.sync_copy(acc_cnt.at[pl.ds(base, ZR)], cnt_hbm.at[c, pl.ds(base, ZR)])


_sc_agg_l1 = pl.kernel(
    _sc_agg_l1_body,
    out_type=(
        jax.ShapeDtypeStruct((NC, NPAD, D // 2), jnp.float32),
        jax.ShapeDtypeStruct((NC, NPAD, L), jnp.float32),
    ),
    mesh=_MESH,
    scratch_types=[
        pltpu.VMEM((CPT, CHUNK), jnp.int32),
        pltpu.VMEM((CPT, CHUNK), jnp.int32),
        pltpu.VMEM((NBUF, CHUNK, D // 2), jnp.float32),
        pltpu.VMEM((CHUNK, L), jnp.float32),
        pltpu.VMEM_SHARED((NPAD, D // 2), jnp.float32),
        pltpu.VMEM_SHARED((NPAD, L), jnp.float32),
        pltpu.SemaphoreType.DMA,
        pltpu.SemaphoreType.DMA,
    ],
    compiler_params=pltpu.CompilerParams(use_tc_tiling_on_sc=False),
)


def _sc_agg_l2_body(t_hbm, src_hbm, dst_hbm, z16_hbm,
                    agg_hbm,
                    src_v, dst_v, rows_v, acc, gsem, ssem):
    c = lax.axis_index("c")
    s = lax.axis_index("s")
    pltpu.sync_copy(z16_hbm, acc.at[pl.ds(s * ZR, ZR)])
    pltpu.sync_copy(src_hbm.at[s], src_v)
    pltpu.sync_copy(dst_hbm.at[s], dst_v)
    plsc.subcore_barrier()
    cph = CPT // 2 + 1 - c
    for b in range(NG2):
        pltpu.async_copy(t_hbm.at[src_v.at[2 * b + c]], rows_v.at[b], gsem)

    def step(t, carry):
        j = 2 * t + c
        pltpu.make_async_copy(t_hbm.at[src_v.at[j]],
                              rows_v.at[t % NBUF2], gsem).wait()

        @pl.when(t >= NK2)
        def _():
            pltpu.make_async_copy(rows_v.at[(t - NK2) % NBUF2],
                                  acc.at[dst_v.at[j - 2 * NK2]], ssem).wait()

        @pl.when(t + NG2 < cph)
        def _():
            pltpu.async_copy(t_hbm.at[src_v.at[j + 2 * NG2]],
                             rows_v.at[(t + NG2) % NBUF2], gsem)

        pltpu.async_copy(rows_v.at[t % NBUF2], acc.at[dst_v.at[j]], ssem,
                         add=True)
        return carry

    lax.fori_loop(0, cph, step, 0)
    for u in range(NK2):
        t = cph - NK2 + u
        pltpu.make_async_copy(rows_v.at[lax.rem(t, NBUF2)],
                              acc.at[dst_v.at[2 * t + c]], ssem).wait()
    plsc.subcore_barrier()
    base = s * ZR
    pltpu.sync_copy(acc.at[pl.ds(base, ZR)], agg_hbm.at[c, pl.ds(base, ZR)])


_sc_agg_l2 = pl.kernel(
    _sc_agg_l2_body,
    out_type=jax.ShapeDtypeStruct((NC, NPAD, L), jnp.float32),
    mesh=_MESH,
    scratch_types=[
        pltpu.VMEM((CPT, CHUNK), jnp.int32),
        pltpu.VMEM((CPT, CHUNK), jnp.int32),
        pltpu.VMEM((NBUF2, CHUNK, L), jnp.float32),
        pltpu.VMEM_SHARED((NPAD, L), jnp.float32),
        pltpu.SemaphoreType.DMA,
        pltpu.SemaphoreType.DMA,
    ],
    compiler_params=pltpu.CompilerParams(use_tc_tiling_on_sc=False),
)


def _matmul_t(a, w):
    return lax.dot_general(a, w, (((1,), (1,)), ((), ())),
                           preferred_element_type=jnp.float32)


BR = 1000


def _tc_layer1_body(aggp, cntp, x, w1l, w1r, b1, w2lp, w2rp, b2p,
                    h2_out, hr_out):
    agg = jnp.concatenate([aggp[0], aggp[1]], axis=1)
    cnt = cntp[0, :, 0:1] + cntp[1, :, 0:1]
    mean = agg / jnp.maximum(cnt, 1.0)
    h = jnp.maximum(_matmul_t(mean, w1l[...]) + b1[...] +
                    _matmul_t(x[...], w1r[...]), 0.0)
    h2_out[...] = _matmul_t(h, w2lp[...])
    hr_out[...] = _matmul_t(h, w2rp[...]) + b2p[...]


def _tc_layer1(agg, cnt, x, w1l, w1r, b1, w2lp, w2rp, b2p):
    grid = N // BR
    full = lambda shape: pl.BlockSpec(shape, lambda i: (0,) * len(shape))
    return pl.pallas_call(
        _tc_layer1_body,
        grid=(grid,),
        in_specs=[
            pl.BlockSpec((NC, BR, D // 2), lambda i: (0, i, 0)),
            pl.BlockSpec((NC, BR, L), lambda i: (0, i, 0)),
            pl.BlockSpec((BR, D), lambda i: (i, 0)),
            full((D, D)),
            full((D, D)),
            full((1, D)),
            full((L, D)),
            full((L, D)),
            full((1, L)),
        ],
        out_specs=[
            pl.BlockSpec((BR, L), lambda i: (i, 0)),
            pl.BlockSpec((BR, L), lambda i: (i, 0)),
        ],
        out_shape=[
            jax.ShapeDtypeStruct((N, L), jnp.float32),
            jax.ShapeDtypeStruct((N, L), jnp.float32),
        ],
    )(agg, cnt, x, w1l, w1r, b1, w2lp, w2rp, b2p)


def _tc_layer2_body(a2p, cntp, hr, out):
    a = a2p[0] + a2p[1]
    cnt = cntp[0, :, 0:1] + cntp[1, :, 0:1]
    mean2 = a / jnp.maximum(cnt, 1.0)
    out[...] = mean2[:, 0:2] + hr[:, 0:2]


def _tc_layer2(agg2, cnt, hr):
    grid = N // BR
    return pl.pallas_call(
        _tc_layer2_body,
        grid=(grid,),
        in_specs=[
            pl.BlockSpec((NC, BR, L), lambda i: (0, i, 0)),
            pl.BlockSpec((NC, BR, L), lambda i: (0, i, 0)),
            pl.BlockSpec((BR, L), lambda i: (i, 0)),
        ],
        out_specs=pl.BlockSpec((BR, 2), lambda i: (i, 0)),
        out_shape=jax.ShapeDtypeStruct((N, 2), jnp.float32),
    )(agg2, cnt, hr)


def kernel(x, edge_index, W1_l, W1_r, b1, W2_l, W2_r, b2):
    src = edge_index[0].astype(jnp.int32)
    dst = edge_index[1].astype(jnp.int32)
    pad_t = E_PAD_T - E
    src_t = jnp.concatenate([src, jnp.zeros((pad_t,), jnp.int32)]
                            ).reshape(NS, CPT, CHUNK)
    dst_t = jnp.concatenate([dst, jnp.full((pad_t,), N, jnp.int32)]
                            ).reshape(NS, CPT, CHUNK)
    z64 = jnp.zeros((ZR, D // 2), jnp.float32)
    z16 = jnp.zeros((ZR, L), jnp.float32)
    ones16 = jnp.ones((CHUNK, L), jnp.float32)
    xlo = x[:, :D // 2]
    xhi = x[:, D // 2:]

    agg, cnt = _sc_agg_l1(xlo, xhi, src_t, dst_t, z64, z16, ones16)

    w2lp = jnp.zeros((L, D), jnp.float32).at[0:2].set(W2_l)
    w2rp = jnp.zeros((L, D), jnp.float32).at[0:2].set(W2_r)
    b2p = jnp.zeros((1, L), jnp.float32).at[0, 0:2].set(b2)
    h2, hr = _tc_layer1(agg, cnt, x, W1_l, W1_r, b1.reshape(1, D),
                        w2lp, w2rp, b2p)

    agg2 = _sc_agg_l2(h2, src_t, dst_t, z16)
    return _tc_layer2(agg2, cnt, hr)
